# baseline (speedup 1.0000x reference)
import jax
import jax.numpy as jnp
import numpy as np
from jax import lax
from jax.experimental import pallas as pl
from jax.experimental.pallas import tpu as pltpu
from jax.experimental.pallas import tpu_sc as plsc

B = 16
S = 16
V = 100000
ROWS = B * S
K = 50
MASK_TOKEN_ID = 1

NC = 2
NS = 16
NWORK = NC * NS

LANES = 16
CV = 2048
NF = 48
LASTB = NF * CV
LASTW = 1664
TAILB = LASTB + LASTW
TAILW = V - TAILB
T0 = np.float32(3.0)
CAP = 512
SPC = CAP + LANES
NEG = np.float32(-np.inf)
NEGTEST = np.float32(-1e38)

TINY = np.float32(np.finfo(np.float32).tiny)
KEY0 = np.uint32(0)
KEY1 = np.uint32(42)


def _iota16():
    return lax.iota(jnp.int32, LANES)


def _key_of(v):
    bits = lax.bitcast_convert_type(v, jnp.uint32)
    sgn = bits >> jnp.uint32(31)
    flip = jnp.where(sgn == jnp.uint32(1), jnp.uint32(0xFFFFFFFF),
                     jnp.uint32(0x80000000))
    return bits ^ flip


def _sc_body(lg_hbm, probs_hbm, tv_hbm, ti_hbm,
             ring0, ring1, zb0, zb1, tin, ztail,
             cand_v, candi_v, topv_v, topi_v, pb_v,
             offs_m, okf_m, klo_m, off2_m,
             si0, si1, so0, so1):
    cid = lax.axis_index("c")
    sid = lax.axis_index("s")
    wid = sid * NC + cid
    r0 = wid * 8
    rings = [ring0, ring1]
    sis = [si0, si1]
    zbs = [zb0, zb1]
    sos = [so0, so1]

    for zb in (zb0, zb1):
        def zinit(t, _, _zb=zb):
            sz = t // 16
            wz = t % 16
            for q in range(8):
                _zb[sz, pl.ds((wz * 8 + q) * LANES, LANES)] = jnp.zeros(
                    (LANES,), jnp.float32)
            return 0
        lax.fori_loop(0, 128, zinit, 0)

    def ztinit(s, _):
        ztail[s, pl.ds(0, LANES)] = jnp.zeros((LANES,), jnp.float32)
        ztail[s, pl.ds(LANES, LANES)] = jnp.zeros((LANES,), jnp.float32)
        return 0
    lax.fori_loop(0, 8, ztinit, 0)

    def tinit(t, _):
        topv_v[pl.ds(t * LANES, LANES)] = jnp.full((LANES,), NEG, jnp.float32)
        topi_v[pl.ds(t * LANES, LANES)] = jnp.zeros((LANES,), jnp.int32)
        return 0
    lax.fori_loop(0, 640 // LANES, tinit, 0)

    def oinit(s, _):
        offs_m[s] = jnp.int32(0)
        return 0
    lax.fori_loop(0, 8, oinit, 0)

    for sl in range(2):
        pltpu.async_copy(lg_hbm.at[pl.ds(r0, 8), pl.ds(sl * CV, CV)],
                         rings[sl], sis[sl])

    def scan_chunk_rows(buf, base, nblk):
        def srow(s, _):
            off0 = jnp.zeros((LANES,), jnp.int32) + offs_m[s]

            def sblk(t, off):
                vs = [buf[s, pl.ds(t * 128 + q * LANES, LANES)]
                      for q in range(8)]
                mx = jnp.maximum(
                    jnp.maximum(jnp.maximum(vs[0], vs[1]),
                                jnp.maximum(vs[2], vs[3])),
                    jnp.maximum(jnp.maximum(vs[4], vs[5]),
                                jnp.maximum(vs[6], vs[7])))
                anyhit = plsc.all_reduce_population_count(mx > T0)[0]

                def hit(off):
                    for q in range(8):
                        v = vs[q]
                        m = v > T0
                        iv = _iota16() + (base + t * 128 + q * LANES)
                        cs = plsc.cumsum(m.astype(jnp.int32))
                        pos = jnp.minimum(off, CAP) + (cs - 1) + s * SPC
                        plsc.store_scatter(cand_v, [pos], v, mask=m)
                        plsc.store_scatter(candi_v, [pos], iv, mask=m)
                        off = off + plsc.all_reduce_population_count(m)
                    return off

                return lax.cond(anyhit > 0, hit, lambda o: o, off)

            offs_m[s] = lax.fori_loop(0, nblk, sblk, off0)[0]
            return 0

        lax.fori_loop(0, 8, srow, 0)

    def p1body(ch, _):
        for sl in range(2):
            ci = ch * 2 + sl
            base = ci * CV
            pltpu.make_async_copy(
                lg_hbm.at[pl.ds(r0, 8), pl.ds(base, CV)], rings[sl],
                sis[sl]).wait()
            scan_chunk_rows(rings[sl], base, 16)

            @pl.when(ci + 2 < NF)
            def _():
                pltpu.async_copy(
                    lg_hbm.at[pl.ds(r0, 8), pl.ds(base + 2 * CV, CV)],
                    rings[sl], sis[sl])
        return 0

    lax.fori_loop(0, NF // 2, p1body, 0)

    pltpu.sync_copy(lg_hbm.at[pl.ds(r0, 8), pl.ds(LASTB, LASTW)],
                    ring0.at[pl.ds(0, 8), pl.ds(0, LASTW)])
    scan_chunk_rows(ring0, LASTB, LASTW // 128)

    pltpu.sync_copy(lg_hbm.at[pl.ds(r0, 8), pl.ds(TAILB, TAILW)], tin)

    def tailrow(s, _):
        off = offs_m[s]
        for q in range(2):
            v = tin[s, pl.ds(q * LANES, LANES)]
            m = v > T0
            iv = _iota16() + (TAILB + q * LANES)
            slot = s * SPC + jnp.minimum(off, CAP)
            plsc.store_compressed(cand_v.at[pl.ds(slot, LANES)], v, mask=m)
            plsc.store_compressed(candi_v.at[pl.ds(slot, LANES)], iv, mask=m)
            off = off + plsc.all_reduce_population_count(m)[0]
        offs_m[s] = off
        return 0

    lax.fori_loop(0, 8, tailrow, 0)

    def rankrow(s, _):
        n = offs_m[s]
        ok = jnp.logical_and(n >= K, n <= CAP)
        okf_m[s] = ok.astype(jnp.int32)

        @pl.when(ok)
        def _():
            sb = s * SPC
            cand_v[pl.ds(sb + n, LANES)] = jnp.full((LANES,), NEG,
                                                    jnp.float32)
            candi_v[pl.ds(sb + n, LANES)] = jnp.zeros((LANES,), jnp.int32)
            nw = (n + LANES - 1) // LANES

            def rank_a(a, off2):
                va = cand_v[pl.ds(sb + a * LANES, LANES)]
                ia = candi_v[pl.ds(sb + a * LANES, LANES)]

                def rank_b(b, accr):
                    vb = cand_v[pl.ds(sb + b * LANES, LANES)]
                    ib = candi_v[pl.ds(sb + b * LANES, LANES)]
                    for l in range(LANES):
                        sv = vb[l]
                        si_ = ib[l]
                        beats = jnp.logical_or(
                            sv > va,
                            jnp.logical_and(sv == va, si_ < ia))
                        accr = accr + beats.astype(jnp.int32)
                    return accr

                accr = lax.fori_loop(0, nw, rank_b,
                                     jnp.zeros((LANES,), jnp.int32))
                member = accr < K
                slot = s * 80 + jnp.minimum(off2, 64)
                plsc.store_compressed(topv_v.at[pl.ds(slot, LANES)], va,
                                      mask=member)
                plsc.store_compressed(topi_v.at[pl.ds(slot, LANES)], ia,
                                      mask=member)
                return off2 + plsc.all_reduce_population_count(member)[0]

            lax.fori_loop(0, nw, rank_a, jnp.int32(0))
        return 0

    lax.fori_loop(0, 8, rankrow, 0)

    def nbad(s, acc):
        return acc + (1 - okf_m[s])

    anybad = lax.fori_loop(0, 8, nbad, jnp.int32(0))

    @pl.when(anybad > 0)
    def _():
        def kinit(s, _):
            klo_m[s] = jnp.uint32(0)
            return 0
        lax.fori_loop(0, 8, kinit, 0)

        def chunk_pass(per_window):
            def one(buf, base, nwin):
                def prow(s, _):
                    def pwin(t, _):
                        v = buf[s, pl.ds(t * LANES, LANES)]
                        per_window(s, v, base + t * LANES)
                        return 0
                    lax.fori_loop(0, nwin, pwin, 0)
                    return 0
                lax.fori_loop(0, 8, prow, 0)

            def cbody(ci, _):
                base = ci * CV
                pltpu.sync_copy(lg_hbm.at[pl.ds(r0, 8), pl.ds(base, CV)],
                                ring0)
                one(ring0, base, CV // LANES)
                return 0

            lax.fori_loop(0, NF, cbody, 0)
            pltpu.sync_copy(lg_hbm.at[pl.ds(r0, 8), pl.ds(LASTB, LASTW)],
                            ring0.at[pl.ds(0, 8), pl.ds(0, LASTW)])
            one(ring0, LASTB, LASTW // LANES)
            one(tin, TAILB, TAILW // LANES)

        def bs_body(i, _):
            bit = jnp.uint32(31) - i.astype(jnp.uint32)

            def cinit(s, _):
                off2_m[s] = jnp.int32(0)
                return 0
            lax.fori_loop(0, 8, cinit, 0)

            def count_win(s, v, vb):
                kk = klo_m[s] | (jnp.uint32(1) << bit)
                c = plsc.all_reduce_population_count(_key_of(v) >= kk)[0]
                off2_m[s] = off2_m[s] + c

            chunk_pass(count_win)

            def kupd(s, _):
                kk = klo_m[s] | (jnp.uint32(1) << bit)
                klo_m[s] = jnp.where(off2_m[s] >= K, kk, klo_m[s])
                return 0
            lax.fori_loop(0, 8, kupd, 0)
            return 0

        lax.fori_loop(0, 32, bs_body, 0)

        def cinit2(s, _):
            off2_m[s] = jnp.int32(0)
            return 0
        lax.fori_loop(0, 8, cinit2, 0)

        for pred_eq in (False, True):
            def coll_win(s, v, vb, _eq=pred_eq):
                kv = _key_of(v)
                tkey = klo_m[s]
                m0 = kv == tkey if _eq else kv > tkey
                m = jnp.logical_and(m0, okf_m[s] == 0)
                iv = _iota16() + vb
                slot = s * 80 + jnp.minimum(off2_m[s], 64)
                plsc.store_compressed(topv_v.at[pl.ds(slot, LANES)], v,
                                      mask=m)
                plsc.store_compressed(topi_v.at[pl.ds(slot, LANES)], iv,
                                      mask=m)
                off2_m[s] = off2_m[s] + \
                    plsc.all_reduce_population_count(m)[0]

            chunk_pass(coll_win)

    def finrow(s, _):
        sb = s * 80
        w48 = topv_v[pl.ds(sb + 48, LANES)]
        topv_v[pl.ds(sb + 48, LANES)] = jnp.where(_iota16() >= 2, NEG, w48)
        wins = [topv_v[pl.ds(sb + w * LANES, LANES)] for w in range(4)]
        macc = jnp.maximum(jnp.maximum(wins[0], wins[1]),
                           jnp.maximum(wins[2], wins[3]))
        ms = jnp.max(macc)
        es = [jnp.exp(wv - ms) for wv in wins]
        zs = jnp.sum(es[0] + es[1] + es[2] + es[3])
        for w in range(4):
            pb_v[pl.ds(sb + w * LANES, LANES)] = es[w] / zs
        pltpu.sync_copy(topv_v.at[pl.ds(sb, 64)],
                        tv_hbm.at[pl.ds((r0 + s) * 64, 64)])
        pltpu.sync_copy(topi_v.at[pl.ds(sb, 64)],
                        ti_hbm.at[pl.ds((r0 + s) * 64, 64)])
        return 0

    lax.fori_loop(0, 8, finrow, 0)

    def scat(zb, base, width, gate):
        def srow(s, _):
            sb = s * 80
            sv = jnp.full((LANES,), 0, jnp.int32) + s
            for w in range(4):
                vw = topv_v[pl.ds(sb + w * LANES, LANES)]
                tiw = topi_v[pl.ds(sb + w * LANES, LANES)]
                pw = pb_v[pl.ds(sb + w * LANES, LANES)]
                m = jnp.logical_and(
                    vw > NEGTEST,
                    jnp.logical_and(tiw >= base, tiw < base + width))
                plsc.store_scatter(zb, [sv, tiw - base], pw * gate, mask=m)
            return 0
        lax.fori_loop(0, 8, srow, 0)

    def p3body(ch, _):
        for sl in range(2):
            ci = ch * 2 + sl
            base = ci * CV

            @pl.when(ci >= 2)
            def _():
                pltpu.make_async_copy(
                    zbs[sl], probs_hbm.at[pl.ds(r0, 8),
                                          pl.ds(base - 2 * CV, CV)],
                    sos[sl]).wait()
                scat(zbs[sl], base - 2 * CV, CV, jnp.float32(0.0))

            scat(zbs[sl], base, CV, jnp.float32(1.0))
            pltpu.async_copy(zbs[sl],
                             probs_hbm.at[pl.ds(r0, 8), pl.ds(base, CV)],
                             sos[sl])
        return 0

    lax.fori_loop(0, NF // 2, p3body, 0)

    for sl in range(2):
        base = (NF - 2 + sl) * CV
        pltpu.make_async_copy(
            zbs[sl], probs_hbm.at[pl.ds(r0, 8), pl.ds(base, CV)],
            sos[sl]).wait()
    scat(zb0, (NF - 2) * CV, CV, jnp.float32(0.0))

    scat(zb0, LASTB, LASTW, jnp.float32(1.0))
    pltpu.sync_copy(zb0.at[pl.ds(0, 8), pl.ds(0, LASTW)],
                    probs_hbm.at[pl.ds(r0, 8), pl.ds(LASTB, LASTW)])

    def tscat(s, _):
        sb = s * 80
        sv = jnp.full((LANES,), 0, jnp.int32) + s
        for w in range(4):
            vw = topv_v[pl.ds(sb + w * LANES, LANES)]
            tiw = topi_v[pl.ds(sb + w * LANES, LANES)]
            pw = pb_v[pl.ds(sb + w * LANES, LANES)]
            m = jnp.logical_and(vw > NEGTEST, tiw >= TAILB)
            plsc.store_scatter(ztail, [sv, tiw - TAILB], pw, mask=m)
        return 0

    lax.fori_loop(0, 8, tscat, 0)
    pltpu.sync_copy(ztail, probs_hbm.at[pl.ds(r0, 8), pl.ds(TAILB, TAILW)])


def _sc_topk_probs(logits2d):
    mesh = plsc.VectorSubcoreMesh(core_axis_name="c", subcore_axis_name="s",
                                  num_cores=NC, num_subcores=NS)
    fn = pl.kernel(
        _sc_body,
        out_type=(
            jax.ShapeDtypeStruct((ROWS, V), jnp.float32),
            jax.ShapeDtypeStruct((ROWS * 64,), jnp.float32),
            jax.ShapeDtypeStruct((ROWS * 64,), jnp.int32),
        ),
        mesh=mesh,
        compiler_params=pltpu.CompilerParams(needs_layout_passes=False,
                                             use_tc_tiling_on_sc=True),
        scratch_types=[
            pltpu.VMEM((8, CV), jnp.float32),
            pltpu.VMEM((8, CV), jnp.float32),
            pltpu.VMEM((8, CV), jnp.float32),
            pltpu.VMEM((8, CV), jnp.float32),
            pltpu.VMEM((8, TAILW), jnp.float32),
            pltpu.VMEM((8, TAILW), jnp.float32),
            pltpu.VMEM((8 * SPC,), jnp.float32),
            pltpu.VMEM((8 * SPC,), jnp.int32),
            pltpu.VMEM((640,), jnp.float32),
            pltpu.VMEM((640,), jnp.int32),
            pltpu.VMEM((640,), jnp.float32),
            pltpu.SMEM((8,), jnp.int32),
            pltpu.SMEM((8,), jnp.int32),
            pltpu.SMEM((8,), jnp.uint32),
            pltpu.SMEM((8,), jnp.int32),
            pltpu.SemaphoreType.DMA,
            pltpu.SemaphoreType.DMA,
            pltpu.SemaphoreType.DMA,
            pltpu.SemaphoreType.DMA,
        ],
    )
    return fn(logits2d)


def _rotl(x, r):
    return (x << jnp.uint32(r)) | (x >> jnp.uint32(32 - r))


def _threefry2x32(x0, x1):
    ks0 = jnp.uint32(KEY0)
    ks1 = jnp.uint32(KEY1)
    ks2 = jnp.uint32(int(KEY0) ^ int(KEY1) ^ 0x1BD11BDA)
    rot_a = (13, 15, 26, 6)
    rot_b = (17, 29, 16, 24)

    x0 = x0 + ks0
    x1 = x1 + ks1

    def rounds(x0, x1, rots):
        for r in rots:
            x0 = x0 + x1
            x1 = _rotl(x1, r)
            x1 = x1 ^ x0
        return x0, x1

    x0, x1 = rounds(x0, x1, rot_a)
    x0 = x0 + ks1
    x1 = x1 + ks2 + jnp.uint32(1)
    x0, x1 = rounds(x0, x1, rot_b)
    x0 = x0 + ks2
    x1 = x1 + ks0 + jnp.uint32(2)
    x0, x1 = rounds(x0, x1, rot_a)
    x0 = x0 + ks0
    x1 = x1 + ks1 + jnp.uint32(3)
    x0, x1 = rounds(x0, x1, rot_b)
    x0 = x0 + ks1
    x1 = x1 + ks2 + jnp.uint32(4)
    x0, x1 = rounds(x0, x1, rot_a)
    x0 = x0 + ks2
    x1 = x1 + ks0 + jnp.uint32(5)
    return x0, x1


def _tc_sample_body(tv_ref, ti_ref, xt_ref, out_ref):
    tv = tv_ref[...]
    ti = ti_ref[...]
    rows = lax.broadcasted_iota(jnp.int32, (ROWS, 64), 0)
    flat = rows * V + ti
    c_lo = flat.astype(jnp.uint32)
    c_hi = jnp.zeros_like(c_lo)
    b0, b1 = _threefry2x32(c_hi, c_lo)
    bits = b0 ^ b1
    fb = (bits >> jnp.uint32(9)) | jnp.uint32(0x3F800000)
    f = lax.bitcast_convert_type(fb, jnp.float32) - jnp.float32(1.0)
    u = f * jnp.float32(np.float32(1.0) - TINY) + TINY
    u = jnp.maximum(TINY, u)
    g = -jnp.log(-jnp.log(u))
    s = tv + g
    m = jnp.max(s, axis=1, keepdims=True)
    lanes = lax.broadcasted_iota(jnp.int32, (ROWS, 64), 1)
    pos = jnp.min(jnp.where(s == m, lanes, 64), axis=1, keepdims=True)
    tok = jnp.sum(jnp.where(lanes == pos, ti, 0), axis=1, keepdims=True)
    xt = xt_ref[...]
    out_ref[...] = jnp.where(xt == MASK_TOKEN_ID, tok, xt)


def _tc_sample(tv, ti, xt):
    return pl.pallas_call(
        _tc_sample_body,
        out_shape=jax.ShapeDtypeStruct((ROWS, 1), jnp.int32),
    )(tv, ti, xt)


def kernel(logits, x_t, top_k):
    del top_k
    l2 = logits.reshape(ROWS, V)
    probs2, tv_flat, ti_flat = _sc_topk_probs(l2)
    tv = tv_flat.reshape(ROWS, 64)
    ti = ti_flat.reshape(ROWS, 64)
    xt = x_t.reshape(ROWS, 1)
    x_out = _tc_sample(tv, ti, xt)
    return x_out.reshape(B, S), probs2.reshape(B, S, V)


# speedup vs baseline: 1.0051x; 1.0051x over previous
import jax
import jax.numpy as jnp
import numpy as np
from jax import lax
from jax.experimental import pallas as pl
from jax.experimental.pallas import tpu as pltpu
from jax.experimental.pallas import tpu_sc as plsc

B = 16
S = 16
V = 100000
ROWS = B * S
K = 50
MASK_TOKEN_ID = 1

NC = 2
NS = 16
NWORK = NC * NS

LANES = 16
CV = 2048
NF = 48
LASTB = NF * CV
LASTW = 1664
TAILB = LASTB + LASTW
TAILW = V - TAILB
T0 = np.float32(3.0)
CAP = 512
SPC = CAP + LANES
NEG = np.float32(-np.inf)
NEGTEST = np.float32(-1e38)

TINY = np.float32(np.finfo(np.float32).tiny)
KEY0 = np.uint32(0)
KEY1 = np.uint32(42)


def _iota16():
    return lax.iota(jnp.int32, LANES)


def _key_of(v):
    bits = lax.bitcast_convert_type(v, jnp.uint32)
    sgn = bits >> jnp.uint32(31)
    flip = jnp.where(sgn == jnp.uint32(1), jnp.uint32(0xFFFFFFFF),
                     jnp.uint32(0x80000000))
    return bits ^ flip


def _sc_body(lg_hbm, probs_hbm, tv_hbm, ti_hbm,
             ring0, ring1, zb0, zb1, tin, ztail,
             cand_v, candi_v, topv_v, topi_v, pb_v,
             offs_m, okf_m, klo_m, off2_m,
             si0, si1, so0, so1):
    cid = lax.axis_index("c")
    sid = lax.axis_index("s")
    wid = sid * NC + cid
    r0 = wid * 8
    rings = [ring0, ring1]
    sis = [si0, si1]
    zbs = [zb0, zb1]
    sos = [so0, so1]

    for zb in (zb0, zb1):
        def zinit(t, _, _zb=zb):
            sz = t // 16
            wz = t % 16
            for q in range(8):
                _zb[sz, pl.ds((wz * 8 + q) * LANES, LANES)] = jnp.zeros(
                    (LANES,), jnp.float32)
            return 0
        lax.fori_loop(0, 128, zinit, 0)

    def ztinit(s, _):
        ztail[s, pl.ds(0, LANES)] = jnp.zeros((LANES,), jnp.float32)
        ztail[s, pl.ds(LANES, LANES)] = jnp.zeros((LANES,), jnp.float32)
        return 0
    lax.fori_loop(0, 8, ztinit, 0)

    def tinit(t, _):
        topv_v[pl.ds(t * LANES, LANES)] = jnp.full((LANES,), NEG, jnp.float32)
        topi_v[pl.ds(t * LANES, LANES)] = jnp.zeros((LANES,), jnp.int32)
        return 0
    lax.fori_loop(0, 640 // LANES, tinit, 0)

    def oinit(s, _):
        offs_m[s] = jnp.int32(0)
        return 0
    lax.fori_loop(0, 8, oinit, 0)

    for sl in range(2):
        pltpu.async_copy(lg_hbm.at[pl.ds(r0, 8), pl.ds(sl * CV, CV)],
                         rings[sl], sis[sl])

    def scan_chunk_rows(buf, base, nblk):
        def srow(s, _):
            off0 = offs_m[s]

            def sblk(t, off):
                vs = [buf[s, pl.ds(t * 128 + q * LANES, LANES)]
                      for q in range(8)]
                mx = jnp.maximum(
                    jnp.maximum(jnp.maximum(vs[0], vs[1]),
                                jnp.maximum(vs[2], vs[3])),
                    jnp.maximum(jnp.maximum(vs[4], vs[5]),
                                jnp.maximum(vs[6], vs[7])))
                anyhit = plsc.all_reduce_population_count(mx > T0)[0]

                def hit(off):
                    for q in range(8):
                        v = vs[q]
                        m = v > T0
                        iv = _iota16() + (base + t * 128 + q * LANES)
                        slot = s * SPC + jnp.minimum(off, CAP)
                        plsc.store_compressed(
                            cand_v.at[pl.ds(slot, LANES)], v, mask=m)
                        plsc.store_compressed(
                            candi_v.at[pl.ds(slot, LANES)], iv, mask=m)
                        off = off + plsc.all_reduce_population_count(m)[0]
                    return off

                return lax.cond(anyhit > 0, hit, lambda o: o, off)

            offs_m[s] = lax.fori_loop(0, nblk, sblk, off0)
            return 0

        lax.fori_loop(0, 8, srow, 0)

    def p1body(ch, _):
        for sl in range(2):
            ci = ch * 2 + sl
            base = ci * CV
            pltpu.make_async_copy(
                lg_hbm.at[pl.ds(r0, 8), pl.ds(base, CV)], rings[sl],
                sis[sl]).wait()
            scan_chunk_rows(rings[sl], base, 16)

            @pl.when(ci + 2 < NF)
            def _():
                pltpu.async_copy(
                    lg_hbm.at[pl.ds(r0, 8), pl.ds(base + 2 * CV, CV)],
                    rings[sl], sis[sl])
        return 0

    lax.fori_loop(0, NF // 2, p1body, 0)

    pltpu.sync_copy(lg_hbm.at[pl.ds(r0, 8), pl.ds(LASTB, LASTW)],
                    ring0.at[pl.ds(0, 8), pl.ds(0, LASTW)])
    scan_chunk_rows(ring0, LASTB, LASTW // 128)

    pltpu.sync_copy(lg_hbm.at[pl.ds(r0, 8), pl.ds(TAILB, TAILW)], tin)

    def tailrow(s, _):
        off = offs_m[s]
        for q in range(2):
            v = tin[s, pl.ds(q * LANES, LANES)]
            m = v > T0
            iv = _iota16() + (TAILB + q * LANES)
            slot = s * SPC + jnp.minimum(off, CAP)
            plsc.store_compressed(cand_v.at[pl.ds(slot, LANES)], v, mask=m)
            plsc.store_compressed(candi_v.at[pl.ds(slot, LANES)], iv, mask=m)
            off = off + plsc.all_reduce_population_count(m)[0]
        offs_m[s] = off
        return 0

    lax.fori_loop(0, 8, tailrow, 0)

    def rankrow(s, _):
        n = offs_m[s]
        ok = jnp.logical_and(n >= K, n <= CAP)
        okf_m[s] = ok.astype(jnp.int32)

        @pl.when(ok)
        def _():
            sb = s * SPC
            cand_v[pl.ds(sb + n, LANES)] = jnp.full((LANES,), NEG,
                                                    jnp.float32)
            candi_v[pl.ds(sb + n, LANES)] = jnp.zeros((LANES,), jnp.int32)
            nw = (n + LANES - 1) // LANES

            def rank_a(a, off2):
                va = cand_v[pl.ds(sb + a * LANES, LANES)]
                ia = candi_v[pl.ds(sb + a * LANES, LANES)]

                def rank_b(b, accr):
                    vb = cand_v[pl.ds(sb + b * LANES, LANES)]
                    ib = candi_v[pl.ds(sb + b * LANES, LANES)]
                    for l in range(LANES):
                        sv = vb[l]
                        si_ = ib[l]
                        beats = jnp.logical_or(
                            sv > va,
                            jnp.logical_and(sv == va, si_ < ia))
                        accr = accr + beats.astype(jnp.int32)
                    return accr

                accr = lax.fori_loop(0, nw, rank_b,
                                     jnp.zeros((LANES,), jnp.int32))
                member = accr < K
                slot = s * 80 + jnp.minimum(off2, 64)
                plsc.store_compressed(topv_v.at[pl.ds(slot, LANES)], va,
                                      mask=member)
                plsc.store_compressed(topi_v.at[pl.ds(slot, LANES)], ia,
                                      mask=member)
                return off2 + plsc.all_reduce_population_count(member)[0]

            lax.fori_loop(0, nw, rank_a, jnp.int32(0))
        return 0

    lax.fori_loop(0, 8, rankrow, 0)

    def nbad(s, acc):
        return acc + (1 - okf_m[s])

    anybad = lax.fori_loop(0, 8, nbad, jnp.int32(0))

    @pl.when(anybad > 0)
    def _():
        def kinit(s, _):
            klo_m[s] = jnp.uint32(0)
            return 0
        lax.fori_loop(0, 8, kinit, 0)

        def chunk_pass(per_window):
            def one(buf, base, nwin):
                def prow(s, _):
                    def pwin(t, _):
                        v = buf[s, pl.ds(t * LANES, LANES)]
                        per_window(s, v, base + t * LANES)
                        return 0
                    lax.fori_loop(0, nwin, pwin, 0)
                    return 0
                lax.fori_loop(0, 8, prow, 0)

            def cbody(ci, _):
                base = ci * CV
                pltpu.sync_copy(lg_hbm.at[pl.ds(r0, 8), pl.ds(base, CV)],
                                ring0)
                one(ring0, base, CV // LANES)
                return 0

            lax.fori_loop(0, NF, cbody, 0)
            pltpu.sync_copy(lg_hbm.at[pl.ds(r0, 8), pl.ds(LASTB, LASTW)],
                            ring0.at[pl.ds(0, 8), pl.ds(0, LASTW)])
            one(ring0, LASTB, LASTW // LANES)
            one(tin, TAILB, TAILW // LANES)

        def bs_body(i, _):
            bit = jnp.uint32(31) - i.astype(jnp.uint32)

            def cinit(s, _):
                off2_m[s] = jnp.int32(0)
                return 0
            lax.fori_loop(0, 8, cinit, 0)

            def count_win(s, v, vb):
                kk = klo_m[s] | (jnp.uint32(1) << bit)
                c = plsc.all_reduce_population_count(_key_of(v) >= kk)[0]
                off2_m[s] = off2_m[s] + c

            chunk_pass(count_win)

            def kupd(s, _):
                kk = klo_m[s] | (jnp.uint32(1) << bit)
                klo_m[s] = jnp.where(off2_m[s] >= K, kk, klo_m[s])
                return 0
            lax.fori_loop(0, 8, kupd, 0)
            return 0

        lax.fori_loop(0, 32, bs_body, 0)

        def cinit2(s, _):
            off2_m[s] = jnp.int32(0)
            return 0
        lax.fori_loop(0, 8, cinit2, 0)

        for pred_eq in (False, True):
            def coll_win(s, v, vb, _eq=pred_eq):
                kv = _key_of(v)
                tkey = klo_m[s]
                m0 = kv == tkey if _eq else kv > tkey
                m = jnp.logical_and(m0, okf_m[s] == 0)
                iv = _iota16() + vb
                slot = s * 80 + jnp.minimum(off2_m[s], 64)
                plsc.store_compressed(topv_v.at[pl.ds(slot, LANES)], v,
                                      mask=m)
                plsc.store_compressed(topi_v.at[pl.ds(slot, LANES)], iv,
                                      mask=m)
                off2_m[s] = off2_m[s] + \
                    plsc.all_reduce_population_count(m)[0]

            chunk_pass(coll_win)

    def finrow(s, _):
        sb = s * 80
        w48 = topv_v[pl.ds(sb + 48, LANES)]
        topv_v[pl.ds(sb + 48, LANES)] = jnp.where(_iota16() >= 2, NEG, w48)
        wins = [topv_v[pl.ds(sb + w * LANES, LANES)] for w in range(4)]
        macc = jnp.maximum(jnp.maximum(wins[0], wins[1]),
                           jnp.maximum(wins[2], wins[3]))
        ms = jnp.max(macc)
        es = [jnp.exp(wv - ms) for wv in wins]
        zs = jnp.sum(es[0] + es[1] + es[2] + es[3])
        for w in range(4):
            pb_v[pl.ds(sb + w * LANES, LANES)] = es[w] / zs
        pltpu.sync_copy(topv_v.at[pl.ds(sb, 64)],
                        tv_hbm.at[pl.ds((r0 + s) * 64, 64)])
        pltpu.sync_copy(topi_v.at[pl.ds(sb, 64)],
                        ti_hbm.at[pl.ds((r0 + s) * 64, 64)])
        return 0

    lax.fori_loop(0, 8, finrow, 0)

    def scat(zb, base, width, gate):
        def srow(s, _):
            sb = s * 80
            sv = jnp.full((LANES,), 0, jnp.int32) + s
            for w in range(4):
                vw = topv_v[pl.ds(sb + w * LANES, LANES)]
                tiw = topi_v[pl.ds(sb + w * LANES, LANES)]
                pw = pb_v[pl.ds(sb + w * LANES, LANES)]
                m = jnp.logical_and(
                    vw > NEGTEST,
                    jnp.logical_and(tiw >= base, tiw < base + width))
                plsc.store_scatter(zb, [sv, tiw - base], pw * gate, mask=m)
            return 0
        lax.fori_loop(0, 8, srow, 0)

    def p3body(ch, _):
        for sl in range(2):
            ci = ch * 2 + sl
            base = ci * CV

            @pl.when(ci >= 2)
            def _():
                pltpu.make_async_copy(
                    zbs[sl], probs_hbm.at[pl.ds(r0, 8),
                                          pl.ds(base - 2 * CV, CV)],
                    sos[sl]).wait()
                scat(zbs[sl], base - 2 * CV, CV, jnp.float32(0.0))

            scat(zbs[sl], base, CV, jnp.float32(1.0))
            pltpu.async_copy(zbs[sl],
                             probs_hbm.at[pl.ds(r0, 8), pl.ds(base, CV)],
                             sos[sl])
        return 0

    lax.fori_loop(0, NF // 2, p3body, 0)

    for sl in range(2):
        base = (NF - 2 + sl) * CV
        pltpu.make_async_copy(
            zbs[sl], probs_hbm.at[pl.ds(r0, 8), pl.ds(base, CV)],
            sos[sl]).wait()
    scat(zb0, (NF - 2) * CV, CV, jnp.float32(0.0))

    scat(zb0, LASTB, LASTW, jnp.float32(1.0))
    pltpu.sync_copy(zb0.at[pl.ds(0, 8), pl.ds(0, LASTW)],
                    probs_hbm.at[pl.ds(r0, 8), pl.ds(LASTB, LASTW)])

    def tscat(s, _):
        sb = s * 80
        sv = jnp.full((LANES,), 0, jnp.int32) + s
        for w in range(4):
            vw = topv_v[pl.ds(sb + w * LANES, LANES)]
            tiw = topi_v[pl.ds(sb + w * LANES, LANES)]
            pw = pb_v[pl.ds(sb + w * LANES, LANES)]
            m = jnp.logical_and(vw > NEGTEST, tiw >= TAILB)
            plsc.store_scatter(ztail, [sv, tiw - TAILB], pw, mask=m)
        return 0

    lax.fori_loop(0, 8, tscat, 0)
    pltpu.sync_copy(ztail, probs_hbm.at[pl.ds(r0, 8), pl.ds(TAILB, TAILW)])


def _sc_topk_probs(logits2d):
    mesh = plsc.VectorSubcoreMesh(core_axis_name="c", subcore_axis_name="s",
                                  num_cores=NC, num_subcores=NS)
    fn = pl.kernel(
        _sc_body,
        out_type=(
            jax.ShapeDtypeStruct((ROWS, V), jnp.float32),
            jax.ShapeDtypeStruct((ROWS * 64,), jnp.float32),
            jax.ShapeDtypeStruct((ROWS * 64,), jnp.int32),
        ),
        mesh=mesh,
        compiler_params=pltpu.CompilerParams(needs_layout_passes=False,
                                             use_tc_tiling_on_sc=True),
        scratch_types=[
            pltpu.VMEM((8, CV), jnp.float32),
            pltpu.VMEM((8, CV), jnp.float32),
            pltpu.VMEM((8, CV), jnp.float32),
            pltpu.VMEM((8, CV), jnp.float32),
            pltpu.VMEM((8, TAILW), jnp.float32),
            pltpu.VMEM((8, TAILW), jnp.float32),
            pltpu.VMEM((8 * SPC,), jnp.float32),
            pltpu.VMEM((8 * SPC,), jnp.int32),
            pltpu.VMEM((640,), jnp.float32),
            pltpu.VMEM((640,), jnp.int32),
            pltpu.VMEM((640,), jnp.float32),
            pltpu.SMEM((8,), jnp.int32),
            pltpu.SMEM((8,), jnp.int32),
            pltpu.SMEM((8,), jnp.uint32),
            pltpu.SMEM((8,), jnp.int32),
            pltpu.SemaphoreType.DMA,
            pltpu.SemaphoreType.DMA,
            pltpu.SemaphoreType.DMA,
            pltpu.SemaphoreType.DMA,
        ],
    )
    return fn(logits2d)


def _rotl(x, r):
    return (x << jnp.uint32(r)) | (x >> jnp.uint32(32 - r))


def _threefry2x32(x0, x1):
    ks0 = jnp.uint32(KEY0)
    ks1 = jnp.uint32(KEY1)
    ks2 = jnp.uint32(int(KEY0) ^ int(KEY1) ^ 0x1BD11BDA)
    rot_a = (13, 15, 26, 6)
    rot_b = (17, 29, 16, 24)

    x0 = x0 + ks0
    x1 = x1 + ks1

    def rounds(x0, x1, rots):
        for r in rots:
            x0 = x0 + x1
            x1 = _rotl(x1, r)
            x1 = x1 ^ x0
        return x0, x1

    x0, x1 = rounds(x0, x1, rot_a)
    x0 = x0 + ks1
    x1 = x1 + ks2 + jnp.uint32(1)
    x0, x1 = rounds(x0, x1, rot_b)
    x0 = x0 + ks2
    x1 = x1 + ks0 + jnp.uint32(2)
    x0, x1 = rounds(x0, x1, rot_a)
    x0 = x0 + ks0
    x1 = x1 + ks1 + jnp.uint32(3)
    x0, x1 = rounds(x0, x1, rot_b)
    x0 = x0 + ks1
    x1 = x1 + ks2 + jnp.uint32(4)
    x0, x1 = rounds(x0, x1, rot_a)
    x0 = x0 + ks2
    x1 = x1 + ks0 + jnp.uint32(5)
    return x0, x1


def _tc_sample_body(tv_ref, ti_ref, xt_ref, out_ref):
    tv = tv_ref[...]
    ti = ti_ref[...]
    rows = lax.broadcasted_iota(jnp.int32, (ROWS, 64), 0)
    flat = rows * V + ti
    c_lo = flat.astype(jnp.uint32)
    c_hi = jnp.zeros_like(c_lo)
    b0, b1 = _threefry2x32(c_hi, c_lo)
    bits = b0 ^ b1
    fb = (bits >> jnp.uint32(9)) | jnp.uint32(0x3F800000)
    f = lax.bitcast_convert_type(fb, jnp.float32) - jnp.float32(1.0)
    u = f * jnp.float32(np.float32(1.0) - TINY) + TINY
    u = jnp.maximum(TINY, u)
    g = -jnp.log(-jnp.log(u))
    s = tv + g
    m = jnp.max(s, axis=1, keepdims=True)
    lanes = lax.broadcasted_iota(jnp.int32, (ROWS, 64), 1)
    pos = jnp.min(jnp.where(s == m, lanes, 64), axis=1, keepdims=True)
    tok = jnp.sum(jnp.where(lanes == pos, ti, 0), axis=1, keepdims=True)
    xt = xt_ref[...]
    out_ref[...] = jnp.where(xt == MASK_TOKEN_ID, tok, xt)


def _tc_sample(tv, ti, xt):
    return pl.pallas_call(
        _tc_sample_body,
        out_shape=jax.ShapeDtypeStruct((ROWS, 1), jnp.int32),
    )(tv, ti, xt)


def kernel(logits, x_t, top_k):
    del top_k
    l2 = logits.reshape(ROWS, V)
    probs2, tv_flat, ti_flat = _sc_topk_probs(l2)
    tv = tv_flat.reshape(ROWS, 64)
    ti = ti_flat.reshape(ROWS, 64)
    xt = x_t.reshape(ROWS, 1)
    x_out = _tc_sample(tv, ti, xt)
    return x_out.reshape(B, S), probs2.reshape(B, S, V)


# speedup vs baseline: 1.0468x; 1.0415x over previous
import jax
import jax.numpy as jnp
import numpy as np
from jax import lax
from jax.experimental import pallas as pl
from jax.experimental.pallas import tpu as pltpu
from jax.experimental.pallas import tpu_sc as plsc

B = 16
S = 16
V = 100000
ROWS = B * S
K = 50
MASK_TOKEN_ID = 1

NC = 2
NS = 16
NWORK = NC * NS

LANES = 16
CV = 2048
NF = 48
LASTB = NF * CV
LASTW = 1664
TAILB = LASTB + LASTW
TAILW = V - TAILB
T0 = np.float32(3.0)
CAP = 512
SPC = CAP + LANES
NEG = np.float32(-np.inf)
NEGTEST = np.float32(-1e38)

TINY = np.float32(np.finfo(np.float32).tiny)
KEY0 = np.uint32(0)
KEY1 = np.uint32(42)


def _iota16():
    return lax.iota(jnp.int32, LANES)


def _key_of(v):
    bits = lax.bitcast_convert_type(v, jnp.uint32)
    sgn = bits >> jnp.uint32(31)
    flip = jnp.where(sgn == jnp.uint32(1), jnp.uint32(0xFFFFFFFF),
                     jnp.uint32(0x80000000))
    return bits ^ flip


def _sc_body(lg_hbm, probs_hbm, tv_hbm, ti_hbm,
             ring0, ring1, zb0, zb1, tin, ztail,
             cand_v, candi_v, topv_v, topi_v, pb_v, rank_v,
             offs_m, okf_m, klo_m, off2_m,
             si0, si1, so0, so1):
    cid = lax.axis_index("c")
    sid = lax.axis_index("s")
    wid = sid * NC + cid
    r0 = wid * 8
    rings = [ring0, ring1]
    sis = [si0, si1]
    zbs = [zb0, zb1]
    sos = [so0, so1]

    for zb in (zb0, zb1):
        def zinit(t, _, _zb=zb):
            sz = t // 16
            wz = t % 16
            for q in range(8):
                _zb[sz, pl.ds((wz * 8 + q) * LANES, LANES)] = jnp.zeros(
                    (LANES,), jnp.float32)
            return 0
        lax.fori_loop(0, 128, zinit, 0)

    def ztinit(s, _):
        ztail[s, pl.ds(0, LANES)] = jnp.zeros((LANES,), jnp.float32)
        ztail[s, pl.ds(LANES, LANES)] = jnp.zeros((LANES,), jnp.float32)
        return 0
    lax.fori_loop(0, 8, ztinit, 0)

    def tinit(t, _):
        topv_v[pl.ds(t * LANES, LANES)] = jnp.full((LANES,), NEG, jnp.float32)
        topi_v[pl.ds(t * LANES, LANES)] = jnp.zeros((LANES,), jnp.int32)
        return 0
    lax.fori_loop(0, 640 // LANES, tinit, 0)

    def oinit(s, _):
        offs_m[s] = jnp.int32(0)
        return 0
    lax.fori_loop(0, 8, oinit, 0)

    for sl in range(2):
        pltpu.async_copy(lg_hbm.at[pl.ds(r0, 8), pl.ds(sl * CV, CV)],
                         rings[sl], sis[sl])

    def scan_chunk_rows(buf, base, nblk):
        def srow(s, _):
            off0 = offs_m[s]

            def sblk(t, off):
                vs = [buf[s, pl.ds(t * 128 + q * LANES, LANES)]
                      for q in range(8)]
                mx = jnp.maximum(
                    jnp.maximum(jnp.maximum(vs[0], vs[1]),
                                jnp.maximum(vs[2], vs[3])),
                    jnp.maximum(jnp.maximum(vs[4], vs[5]),
                                jnp.maximum(vs[6], vs[7])))
                anyhit = plsc.all_reduce_population_count(mx > T0)[0]

                def hit(off):
                    for q in range(8):
                        v = vs[q]
                        m = v > T0
                        iv = _iota16() + (base + t * 128 + q * LANES)
                        slot = s * SPC + jnp.minimum(off, CAP)
                        plsc.store_compressed(
                            cand_v.at[pl.ds(slot, LANES)], v, mask=m)
                        plsc.store_compressed(
                            candi_v.at[pl.ds(slot, LANES)], iv, mask=m)
                        off = off + plsc.all_reduce_population_count(m)[0]
                    return off

                return lax.cond(anyhit > 0, hit, lambda o: o, off)

            offs_m[s] = lax.fori_loop(0, nblk, sblk, off0)
            return 0

        lax.fori_loop(0, 8, srow, 0)

    def p1body(ch, _):
        for sl in range(2):
            ci = ch * 2 + sl
            base = ci * CV
            pltpu.make_async_copy(
                lg_hbm.at[pl.ds(r0, 8), pl.ds(base, CV)], rings[sl],
                sis[sl]).wait()
            scan_chunk_rows(rings[sl], base, 16)

            @pl.when(ci + 2 < NF)
            def _():
                pltpu.async_copy(
                    lg_hbm.at[pl.ds(r0, 8), pl.ds(base + 2 * CV, CV)],
                    rings[sl], sis[sl])
        return 0

    lax.fori_loop(0, NF // 2, p1body, 0)

    pltpu.sync_copy(lg_hbm.at[pl.ds(r0, 8), pl.ds(LASTB, LASTW)],
                    ring0.at[pl.ds(0, 8), pl.ds(0, LASTW)])
    scan_chunk_rows(ring0, LASTB, LASTW // 128)

    pltpu.sync_copy(lg_hbm.at[pl.ds(r0, 8), pl.ds(TAILB, TAILW)], tin)

    def tailrow(s, _):
        off = offs_m[s]
        for q in range(2):
            v = tin[s, pl.ds(q * LANES, LANES)]
            m = v > T0
            iv = _iota16() + (TAILB + q * LANES)
            slot = s * SPC + jnp.minimum(off, CAP)
            plsc.store_compressed(cand_v.at[pl.ds(slot, LANES)], v, mask=m)
            plsc.store_compressed(candi_v.at[pl.ds(slot, LANES)], iv, mask=m)
            off = off + plsc.all_reduce_population_count(m)[0]
        offs_m[s] = off
        return 0

    lax.fori_loop(0, 8, tailrow, 0)

    def rankrow(s, _):
        n = offs_m[s]
        ok = jnp.logical_and(n >= K, n <= CAP)
        okf_m[s] = ok.astype(jnp.int32)

        @pl.when(ok)
        def _():
            sb = s * SPC
            cand_v[pl.ds(sb + n, LANES)] = jnp.full((LANES,), NEG,
                                                    jnp.float32)
            candi_v[pl.ds(sb + n, LANES)] = jnp.zeros((LANES,), jnp.int32)
            nw = (n + LANES - 1) // LANES

            def fast_a(a, total):
                va = cand_v[pl.ds(sb + a * LANES, LANES)]

                def fast_b(b, accr):
                    vb = cand_v[pl.ds(sb + b * LANES, LANES)]
                    for l in range(LANES):
                        accr = accr + (vb[l] > va).astype(jnp.int32)
                    return accr

                accr = lax.fori_loop(0, nw, fast_b,
                                     jnp.zeros((LANES,), jnp.int32))
                rank_v[pl.ds(a * LANES, LANES)] = accr
                return total + plsc.all_reduce_population_count(
                    accr < K)[0]

            total = lax.fori_loop(0, nw, fast_a, jnp.int32(0))

            def fast_append(_):
                def app_a(a, off2):
                    va = cand_v[pl.ds(sb + a * LANES, LANES)]
                    ia = candi_v[pl.ds(sb + a * LANES, LANES)]
                    member = rank_v[pl.ds(a * LANES, LANES)] < K
                    slot = s * 80 + jnp.minimum(off2, 64)
                    plsc.store_compressed(topv_v.at[pl.ds(slot, LANES)],
                                          va, mask=member)
                    plsc.store_compressed(topi_v.at[pl.ds(slot, LANES)],
                                          ia, mask=member)
                    return off2 + plsc.all_reduce_population_count(
                        member)[0]

                lax.fori_loop(0, nw, app_a, jnp.int32(0))
                return 0

            def precise(_):
                def rank_a(a, off2):
                    va = cand_v[pl.ds(sb + a * LANES, LANES)]
                    ia = candi_v[pl.ds(sb + a * LANES, LANES)]

                    def rank_b(b, accr):
                        vb = cand_v[pl.ds(sb + b * LANES, LANES)]
                        ib = candi_v[pl.ds(sb + b * LANES, LANES)]
                        for l in range(LANES):
                            sv = vb[l]
                            si_ = ib[l]
                            beats = jnp.logical_or(
                                sv > va,
                                jnp.logical_and(sv == va, si_ < ia))
                            accr = accr + beats.astype(jnp.int32)
                        return accr

                    accr = lax.fori_loop(0, nw, rank_b,
                                         jnp.zeros((LANES,), jnp.int32))
                    member = accr < K
                    slot = s * 80 + jnp.minimum(off2, 64)
                    plsc.store_compressed(topv_v.at[pl.ds(slot, LANES)],
                                          va, mask=member)
                    plsc.store_compressed(topi_v.at[pl.ds(slot, LANES)],
                                          ia, mask=member)
                    return off2 + plsc.all_reduce_population_count(
                        member)[0]

                lax.fori_loop(0, nw, rank_a, jnp.int32(0))
                return 0

            lax.cond(total == K, fast_append, precise, 0)
        return 0

    lax.fori_loop(0, 8, rankrow, 0)

    def nbad(s, acc):
        return acc + (1 - okf_m[s])

    anybad = lax.fori_loop(0, 8, nbad, jnp.int32(0))

    @pl.when(anybad > 0)
    def _():
        def kinit(s, _):
            klo_m[s] = jnp.uint32(0)
            return 0
        lax.fori_loop(0, 8, kinit, 0)

        def chunk_pass(per_window):
            def one(buf, base, nwin):
                def prow(s, _):
                    def pwin(t, _):
                        v = buf[s, pl.ds(t * LANES, LANES)]
                        per_window(s, v, base + t * LANES)
                        return 0
                    lax.fori_loop(0, nwin, pwin, 0)
                    return 0
                lax.fori_loop(0, 8, prow, 0)

            def cbody(ci, _):
                base = ci * CV
                pltpu.sync_copy(lg_hbm.at[pl.ds(r0, 8), pl.ds(base, CV)],
                                ring0)
                one(ring0, base, CV // LANES)
                return 0

            lax.fori_loop(0, NF, cbody, 0)
            pltpu.sync_copy(lg_hbm.at[pl.ds(r0, 8), pl.ds(LASTB, LASTW)],
                            ring0.at[pl.ds(0, 8), pl.ds(0, LASTW)])
            one(ring0, LASTB, LASTW // LANES)
            one(tin, TAILB, TAILW // LANES)

        def bs_body(i, _):
            bit = jnp.uint32(31) - i.astype(jnp.uint32)

            def cinit(s, _):
                off2_m[s] = jnp.int32(0)
                return 0
            lax.fori_loop(0, 8, cinit, 0)

            def count_win(s, v, vb):
                kk = klo_m[s] | (jnp.uint32(1) << bit)
                c = plsc.all_reduce_population_count(_key_of(v) >= kk)[0]
                off2_m[s] = off2_m[s] + c

            chunk_pass(count_win)

            def kupd(s, _):
                kk = klo_m[s] | (jnp.uint32(1) << bit)
                klo_m[s] = jnp.where(off2_m[s] >= K, kk, klo_m[s])
                return 0
            lax.fori_loop(0, 8, kupd, 0)
            return 0

        lax.fori_loop(0, 32, bs_body, 0)

        def cinit2(s, _):
            off2_m[s] = jnp.int32(0)
            return 0
        lax.fori_loop(0, 8, cinit2, 0)

        for pred_eq in (False, True):
            def coll_win(s, v, vb, _eq=pred_eq):
                kv = _key_of(v)
                tkey = klo_m[s]
                m0 = kv == tkey if _eq else kv > tkey
                m = jnp.logical_and(m0, okf_m[s] == 0)
                iv = _iota16() + vb
                slot = s * 80 + jnp.minimum(off2_m[s], 64)
                plsc.store_compressed(topv_v.at[pl.ds(slot, LANES)], v,
                                      mask=m)
                plsc.store_compressed(topi_v.at[pl.ds(slot, LANES)], iv,
                                      mask=m)
                off2_m[s] = off2_m[s] + \
                    plsc.all_reduce_population_count(m)[0]

            chunk_pass(coll_win)

    def finrow(s, _):
        sb = s * 80
        w48 = topv_v[pl.ds(sb + 48, LANES)]
        topv_v[pl.ds(sb + 48, LANES)] = jnp.where(_iota16() >= 2, NEG, w48)
        wins = [topv_v[pl.ds(sb + w * LANES, LANES)] for w in range(4)]
        macc = jnp.maximum(jnp.maximum(wins[0], wins[1]),
                           jnp.maximum(wins[2], wins[3]))
        ms = jnp.max(macc)
        es = [jnp.exp(wv - ms) for wv in wins]
        zs = jnp.sum(es[0] + es[1] + es[2] + es[3])
        for w in range(4):
            pb_v[pl.ds(sb + w * LANES, LANES)] = es[w] / zs
        pltpu.sync_copy(topv_v.at[pl.ds(sb, 64)],
                        tv_hbm.at[pl.ds((r0 + s) * 64, 64)])
        pltpu.sync_copy(topi_v.at[pl.ds(sb, 64)],
                        ti_hbm.at[pl.ds((r0 + s) * 64, 64)])
        return 0

    lax.fori_loop(0, 8, finrow, 0)

    def scat(zb, base, width, gate):
        def srow(s, _):
            sb = s * 80
            sv = jnp.full((LANES,), 0, jnp.int32) + s
            for w in range(4):
                vw = topv_v[pl.ds(sb + w * LANES, LANES)]
                tiw = topi_v[pl.ds(sb + w * LANES, LANES)]
                pw = pb_v[pl.ds(sb + w * LANES, LANES)]
                m = jnp.logical_and(
                    vw > NEGTEST,
                    jnp.logical_and(tiw >= base, tiw < base + width))
                plsc.store_scatter(zb, [sv, tiw - base], pw * gate, mask=m)
            return 0
        lax.fori_loop(0, 8, srow, 0)

    def p3body(ch, _):
        for sl in range(2):
            ci = ch * 2 + sl
            base = ci * CV

            @pl.when(ci >= 2)
            def _():
                pltpu.make_async_copy(
                    zbs[sl], probs_hbm.at[pl.ds(r0, 8),
                                          pl.ds(base - 2 * CV, CV)],
                    sos[sl]).wait()
                scat(zbs[sl], base - 2 * CV, CV, jnp.float32(0.0))

            scat(zbs[sl], base, CV, jnp.float32(1.0))
            pltpu.async_copy(zbs[sl],
                             probs_hbm.at[pl.ds(r0, 8), pl.ds(base, CV)],
                             sos[sl])
        return 0

    lax.fori_loop(0, NF // 2, p3body, 0)

    for sl in range(2):
        base = (NF - 2 + sl) * CV
        pltpu.make_async_copy(
            zbs[sl], probs_hbm.at[pl.ds(r0, 8), pl.ds(base, CV)],
            sos[sl]).wait()
    scat(zb0, (NF - 2) * CV, CV, jnp.float32(0.0))

    scat(zb0, LASTB, LASTW, jnp.float32(1.0))
    pltpu.sync_copy(zb0.at[pl.ds(0, 8), pl.ds(0, LASTW)],
                    probs_hbm.at[pl.ds(r0, 8), pl.ds(LASTB, LASTW)])

    def tscat(s, _):
        sb = s * 80
        sv = jnp.full((LANES,), 0, jnp.int32) + s
        for w in range(4):
            vw = topv_v[pl.ds(sb + w * LANES, LANES)]
            tiw = topi_v[pl.ds(sb + w * LANES, LANES)]
            pw = pb_v[pl.ds(sb + w * LANES, LANES)]
            m = jnp.logical_and(vw > NEGTEST, tiw >= TAILB)
            plsc.store_scatter(ztail, [sv, tiw - TAILB], pw, mask=m)
        return 0

    lax.fori_loop(0, 8, tscat, 0)
    pltpu.sync_copy(ztail, probs_hbm.at[pl.ds(r0, 8), pl.ds(TAILB, TAILW)])


def _sc_topk_probs(logits2d):
    mesh = plsc.VectorSubcoreMesh(core_axis_name="c", subcore_axis_name="s",
                                  num_cores=NC, num_subcores=NS)
    fn = pl.kernel(
        _sc_body,
        out_type=(
            jax.ShapeDtypeStruct((ROWS, V), jnp.float32),
            jax.ShapeDtypeStruct((ROWS * 64,), jnp.float32),
            jax.ShapeDtypeStruct((ROWS * 64,), jnp.int32),
        ),
        mesh=mesh,
        compiler_params=pltpu.CompilerParams(needs_layout_passes=False,
                                             use_tc_tiling_on_sc=True),
        scratch_types=[
            pltpu.VMEM((8, CV), jnp.float32),
            pltpu.VMEM((8, CV), jnp.float32),
            pltpu.VMEM((8, CV), jnp.float32),
            pltpu.VMEM((8, CV), jnp.float32),
            pltpu.VMEM((8, TAILW), jnp.float32),
            pltpu.VMEM((8, TAILW), jnp.float32),
            pltpu.VMEM((8 * SPC,), jnp.float32),
            pltpu.VMEM((8 * SPC,), jnp.int32),
            pltpu.VMEM((640,), jnp.float32),
            pltpu.VMEM((640,), jnp.int32),
            pltpu.VMEM((640,), jnp.float32),
            pltpu.VMEM((SPC,), jnp.int32),
            pltpu.SMEM((8,), jnp.int32),
            pltpu.SMEM((8,), jnp.int32),
            pltpu.SMEM((8,), jnp.uint32),
            pltpu.SMEM((8,), jnp.int32),
            pltpu.SemaphoreType.DMA,
            pltpu.SemaphoreType.DMA,
            pltpu.SemaphoreType.DMA,
            pltpu.SemaphoreType.DMA,
        ],
    )
    return fn(logits2d)


def _rotl(x, r):
    return (x << jnp.uint32(r)) | (x >> jnp.uint32(32 - r))


def _threefry2x32(x0, x1):
    ks0 = jnp.uint32(KEY0)
    ks1 = jnp.uint32(KEY1)
    ks2 = jnp.uint32(int(KEY0) ^ int(KEY1) ^ 0x1BD11BDA)
    rot_a = (13, 15, 26, 6)
    rot_b = (17, 29, 16, 24)

    x0 = x0 + ks0
    x1 = x1 + ks1

    def rounds(x0, x1, rots):
        for r in rots:
            x0 = x0 + x1
            x1 = _rotl(x1, r)
            x1 = x1 ^ x0
        return x0, x1

    x0, x1 = rounds(x0, x1, rot_a)
    x0 = x0 + ks1
    x1 = x1 + ks2 + jnp.uint32(1)
    x0, x1 = rounds(x0, x1, rot_b)
    x0 = x0 + ks2
    x1 = x1 + ks0 + jnp.uint32(2)
    x0, x1 = rounds(x0, x1, rot_a)
    x0 = x0 + ks0
    x1 = x1 + ks1 + jnp.uint32(3)
    x0, x1 = rounds(x0, x1, rot_b)
    x0 = x0 + ks1
    x1 = x1 + ks2 + jnp.uint32(4)
    x0, x1 = rounds(x0, x1, rot_a)
    x0 = x0 + ks2
    x1 = x1 + ks0 + jnp.uint32(5)
    return x0, x1


def _tc_sample_body(tv_ref, ti_ref, xt_ref, out_ref):
    tv = tv_ref[...]
    ti = ti_ref[...]
    rows = lax.broadcasted_iota(jnp.int32, (ROWS, 64), 0)
    flat = rows * V + ti
    c_lo = flat.astype(jnp.uint32)
    c_hi = jnp.zeros_like(c_lo)
    b0, b1 = _threefry2x32(c_hi, c_lo)
    bits = b0 ^ b1
    fb = (bits >> jnp.uint32(9)) | jnp.uint32(0x3F800000)
    f = lax.bitcast_convert_type(fb, jnp.float32) - jnp.float32(1.0)
    u = f * jnp.float32(np.float32(1.0) - TINY) + TINY
    u = jnp.maximum(TINY, u)
    g = -jnp.log(-jnp.log(u))
    s = tv + g
    m = jnp.max(s, axis=1, keepdims=True)
    lanes = lax.broadcasted_iota(jnp.int32, (ROWS, 64), 1)
    pos = jnp.min(jnp.where(s == m, lanes, 64), axis=1, keepdims=True)
    tok = jnp.sum(jnp.where(lanes == pos, ti, 0), axis=1, keepdims=True)
    xt = xt_ref[...]
    out_ref[...] = jnp.where(xt == MASK_TOKEN_ID, tok, xt)


def _tc_sample(tv, ti, xt):
    return pl.pallas_call(
        _tc_sample_body,
        out_shape=jax.ShapeDtypeStruct((ROWS, 1), jnp.int32),
    )(tv, ti, xt)


def kernel(logits, x_t, top_k):
    del top_k
    l2 = logits.reshape(ROWS, V)
    probs2, tv_flat, ti_flat = _sc_topk_probs(l2)
    tv = tv_flat.reshape(ROWS, 64)
    ti = ti_flat.reshape(ROWS, 64)
    xt = x_t.reshape(ROWS, 1)
    x_out = _tc_sample(tv, ti, xt)
    return x_out.reshape(B, S), probs2.reshape(B, S, V)
